# baseline (speedup 1.0000x reference)
import functools

import jax
import jax.numpy as jnp
from jax import lax
from jax.experimental import pallas as pl
from jax.experimental.pallas import tpu as pltpu
from jax.experimental.pallas import tpu_sc as plsc


def _sc_gather(x, emb):
    (B,) = x.shape
    V, D = emb.shape
    info = plsc.get_sparse_core_info()
    nw = info.num_cores * info.num_subcores
    b_per_w = B // nw

    mesh = plsc.VectorSubcoreMesh(core_axis_name="c", subcore_axis_name="s")

    @functools.partial(
        pl.kernel,
        mesh=mesh,
        out_type=jax.ShapeDtypeStruct((B, D), jnp.float32),
        compiler_params=pltpu.CompilerParams(use_tc_tiling_on_sc=False),
        scratch_types=[
            pltpu.VMEM((b_per_w,), jnp.int32),
            pltpu.VMEM((b_per_w, D), jnp.float32),
            pltpu.SemaphoreType.DMA,
        ],
    )
    def gather_k(idx_hbm, table_hbm, out_hbm, idx_v, rows_v, sem):
        wid = lax.axis_index("s") * info.num_cores + lax.axis_index("c")
        base = wid * b_per_w
        pltpu.sync_copy(idx_hbm.at[pl.ds(base, b_per_w)], idx_v)
        pltpu.async_copy(table_hbm.at[idx_v], rows_v, sem).wait()
        pltpu.sync_copy(rows_v, out_hbm.at[pl.ds(base, b_per_w)])

    return gather_k(x, emb)


def _tc_project(h, W, b2d, vt, nbuf):
    B, D = h.shape
    V = W.shape[0]
    nfull = V // vt
    rem = V - nfull * vt
    grid = nfull + (1 if rem else 0)

    def body(h_ref, w_ref, b_ref, out_hbm, bufs, tail_buf, sems):
        i = pl.program_id(0)
        n = pl.num_programs(0)
        slot = lax.rem(i, nbuf)

        def copy_full(step, s):
            return pltpu.make_async_copy(
                bufs.at[s],
                out_hbm.at[:, pl.ds(step * vt, vt)],
                sems.at[s],
            )

        def copy_tail(s):
            return pltpu.make_async_copy(
                tail_buf,
                out_hbm.at[:, pl.ds(nfull * vt, rem)],
                sems.at[s],
            )

        @pl.when(i >= nbuf)
        def _():
            copy_full(i - nbuf, slot).wait()

        val = (
            lax.dot_general(
                h_ref[...],
                w_ref[...],
                dimension_numbers=(((1,), (1,)), ((), ())),
                preferred_element_type=jnp.float32,
            )
            + b_ref[...]
        )

        if rem:
            @pl.when(i < nfull)
            def _():
                bufs[slot] = val
                copy_full(i, slot).start()

            @pl.when(i == nfull)
            def _():
                tail_buf[...] = val[:, :rem]
                copy_tail(slot).start()
        else:
            bufs[slot] = val
            copy_full(i, slot).start()

        @pl.when(i == n - 1)
        def _():
            for k in range(min(nbuf, grid)):
                step = grid - 1 - k
                if rem and step == nfull:
                    copy_tail(step % nbuf).wait()
                else:
                    copy_full(step, step % nbuf).wait()

    return pl.pallas_call(
        body,
        grid=(grid,),
        in_specs=[
            pl.BlockSpec((B, D), lambda i: (0, 0)),
            pl.BlockSpec((vt, D), lambda i: (i, 0)),
            pl.BlockSpec((1, vt), lambda i: (0, i)),
        ],
        out_specs=pl.BlockSpec(memory_space=pl.ANY),
        out_shape=jax.ShapeDtypeStruct((B, V), jnp.float32),
        scratch_shapes=[
            pltpu.VMEM((nbuf, B, vt), jnp.float32),
            pltpu.VMEM((B, rem if rem else vt), jnp.float32),
            pltpu.SemaphoreType.DMA((nbuf,)),
        ],
    )(h, W, b2d)


def kernel(x, emb, W, b):
    h = _sc_gather(x.astype(jnp.int32), emb)
    return _tc_project(h, W, b.reshape(1, -1), vt=2048, nbuf=4)


# speedup vs baseline: 3.0412x; 3.0412x over previous
import functools

import jax
import jax.numpy as jnp
from jax import lax
from jax.experimental import pallas as pl
from jax.experimental.pallas import tpu as pltpu
from jax.experimental.pallas import tpu_sc as plsc


def _sc_gather(x, emb):
    (B,) = x.shape
    V, D = emb.shape
    info = plsc.get_sparse_core_info()
    nw = info.num_cores * info.num_subcores
    b_per_w = B // nw

    mesh = plsc.VectorSubcoreMesh(core_axis_name="c", subcore_axis_name="s")

    @functools.partial(
        pl.kernel,
        mesh=mesh,
        out_type=jax.ShapeDtypeStruct((B, D), jnp.float32),
        compiler_params=pltpu.CompilerParams(use_tc_tiling_on_sc=False),
        scratch_types=[
            pltpu.VMEM((b_per_w,), jnp.int32),
            pltpu.VMEM((b_per_w, D), jnp.float32),
            pltpu.SemaphoreType.DMA,
        ],
    )
    def gather_k(idx_hbm, table_hbm, out_hbm, idx_v, rows_v, sem):
        wid = lax.axis_index("s") * info.num_cores + lax.axis_index("c")
        base = wid * b_per_w
        pltpu.sync_copy(idx_hbm.at[pl.ds(base, b_per_w)], idx_v)
        pltpu.async_copy(table_hbm.at[idx_v], rows_v, sem).wait()
        pltpu.sync_copy(rows_v, out_hbm.at[pl.ds(base, b_per_w)])

    return gather_k(x, emb)


def _tc_project_t(h, Wt, brow, vt, nbuf):
    B, D = h.shape
    V = Wt.shape[1]
    nfull = V // vt
    rem = V - nfull * vt
    grid = nfull + (1 if rem else 0)

    def body(h_ref, wt_ref, b_ref, out_hbm, bufs, sems):
        i = pl.program_id(0)
        n = pl.num_programs(0)
        slot = lax.rem(i, nbuf)

        def copy_for(step, s, width):
            return pltpu.make_async_copy(
                bufs.at[s, pl.ds(0, width), :],
                out_hbm.at[pl.ds(step * vt, width), :],
                sems.at[s],
            )

        @pl.when(i >= nbuf)
        def _():
            copy_for(i - nbuf, slot, vt).wait()

        val = lax.dot_general(
            wt_ref[...],
            h_ref[...],
            dimension_numbers=(((0,), (1,)), ((), ())),
            preferred_element_type=jnp.float32,
        ) + jnp.transpose(b_ref[...], (1, 0))
        bufs[slot] = val

        if rem:
            @pl.when(i < nfull)
            def _():
                copy_for(i, slot, vt).start()

            @pl.when(i == nfull)
            def _():
                copy_for(nfull, slot, rem).start()
        else:
            copy_for(i, slot, vt).start()

        @pl.when(i == n - 1)
        def _():
            for k in range(min(nbuf, grid)):
                step = grid - 1 - k
                width = rem if (rem and step == nfull) else vt
                copy_for(step, step % nbuf, width).wait()

    return pl.pallas_call(
        body,
        grid=(grid,),
        in_specs=[
            pl.BlockSpec((B, D), lambda i: (0, 0)),
            pl.BlockSpec((D, vt), lambda i: (0, i)),
            pl.BlockSpec((1, vt), lambda i: (0, i)),
        ],
        out_specs=pl.BlockSpec(memory_space=pl.ANY),
        out_shape=jax.ShapeDtypeStruct((V, B), jnp.float32),
        scratch_shapes=[
            pltpu.VMEM((nbuf, vt, B), jnp.float32),
            pltpu.SemaphoreType.DMA((nbuf,)),
        ],
    )(h, Wt, brow)


def kernel(x, emb, W, b):
    h = _sc_gather(x.astype(jnp.int32), emb)
    out_t = _tc_project_t(h, W.T, b.reshape(1, -1), vt=2048, nbuf=4)
    return out_t.T
